# baseline (speedup 1.0000x reference)
import functools

import jax
import jax.numpy as jnp
from jax import lax
from jax.experimental import pallas as pl
from jax.experimental.pallas import tpu as pltpu

SEQ = 4096
NB = 64
NC = 128
UNROLL = 32


def _scan_kernel(z_ref, whh_ref, o_ref, zb0, zb1, *, seq, unroll):
    u = unroll
    ng = seq // u
    w_i = whh_ref[0]
    w_f = whh_ref[1]
    w_g = whh_ref[2]
    w_o = whh_ref[3]

    def stage(g, buf):
        gc = jnp.minimum(g, ng - 1)
        buf[...] = z_ref[pl.ds(gc * u, u), :, :].reshape(u, 4, 1, NC)

    def steps(buf, st):
        h, c = st
        hs = []
        blocks = []
        for j in range(u):
            tg = jnp.tanh(buf[j, 2] + h * w_g)
            ti = jnp.tanh(buf[j, 0] + h * w_i)
            tf = jnp.tanh(buf[j, 1] + h * w_f)
            to = jnp.tanh(buf[j, 3] + h * w_o)
            ig = ti * 0.5 + 0.5
            fg = tf * 0.5 + 0.5
            og = to * 0.5 + 0.5
            c = fg * c + ig * tg
            h = og * jnp.tanh(c)
            hs.append(h)
            if len(hs) == 8:
                blocks.append(jnp.concatenate(hs, axis=0))
                hs = []
        return blocks, (h, c)

    def half(g, buf_run, buf_next, st):
        stage(g + 1, buf_next)
        blocks, st = steps(buf_run, st)
        base = g * u
        for k, blk in enumerate(blocks):
            o_ref[pl.ds(base + 8 * k, 8), :] = blk
        return st

    def body(gg, st):
        g = gg * 2
        st = half(g, zb0, zb1, st)
        st = half(g + 1, zb1, zb0, st)
        return st

    zv = jnp.zeros((1, NC), jnp.float32)
    stage(0, zb0)
    lax.fori_loop(0, ng // 2, body, (zv, zv))


def _mlp_kernel(x_ref, w1_ref, b1_ref, w2_ref, b2_ref, w3_ref, b3_ref,
                o_ref, acc_ref, *, kt):
    k = pl.program_id(0)

    @pl.when(k == 0)
    def _():
        acc_ref[...] = jnp.zeros_like(acc_ref)

    acc_ref[...] += jnp.dot(x_ref[...], w1_ref[...],
                            preferred_element_type=jnp.float32)

    @pl.when(k == kt - 1)
    def _():
        h1 = acc_ref[...] + b1_ref[...]
        h2 = jnp.maximum(
            jnp.dot(h1, w2_ref[...], preferred_element_type=jnp.float32)
            + b2_ref[...], 0.0)
        y = jnp.dot(h2, w3_ref[...],
                    preferred_element_type=jnp.float32) + b3_ref[...]
        o_ref[...] = y


_GATE_SCALE = (0.5, 0.5, 1.0, 0.5)


def _layer_scan(x_seq, wih_f, whh_f, bih_f, bhh_f, wih_b, whh_b, bih_b,
                bhh_b):
    seq = x_seq.shape[1]
    gsc = jnp.array(_GATE_SCALE, jnp.float32)

    def preact(w_ih, b_ih, b_hh, xs):
        z = jnp.einsum("btd,gd->tgb", xs, w_ih)
        return z + (b_ih + b_hh)[None, :, None]

    z = jnp.concatenate(
        [preact(wih_f, bih_f, bhh_f, x_seq),
         preact(wih_b, bih_b, bhh_b, x_seq[:, ::-1, :])], axis=-1)
    z = (z * gsc[None, :, None]).astype(jnp.float32)
    whh = jnp.concatenate(
        [jnp.tile(whh_f, (1, NB)), jnp.tile(whh_b, (1, NB))], axis=1)
    whh = (whh * gsc[:, None]).astype(jnp.float32)

    return pl.pallas_call(
        functools.partial(_scan_kernel, seq=seq, unroll=UNROLL),
        out_shape=jax.ShapeDtypeStruct((seq, NC), jnp.float32),
        in_specs=[pl.BlockSpec(memory_space=pltpu.MemorySpace.VMEM)] * 2,
        out_specs=pl.BlockSpec(memory_space=pltpu.MemorySpace.VMEM),
        scratch_shapes=[pltpu.VMEM((UNROLL, 4, 1, NC), jnp.float32),
                        pltpu.VMEM((UNROLL, 4, 1, NC), jnp.float32)],
        compiler_params=pltpu.CompilerParams(
            vmem_limit_bytes=32 * 1024 * 1024),
    )(z, whh.reshape(4, 1, NC))


def _unscan(h_seq):
    hf = jnp.transpose(h_seq[:, 0:NB])
    hb = jnp.transpose(h_seq[::-1, NB:NC])
    return jnp.stack([hf, hb], axis=-1)


def kernel(pos, batch,
           lstm_0_f_w_ih, lstm_0_f_w_hh, lstm_0_f_b_ih, lstm_0_f_b_hh,
           lstm_0_b_w_ih, lstm_0_b_w_hh, lstm_0_b_b_ih, lstm_0_b_b_hh,
           lstm_1_f_w_ih, lstm_1_f_w_hh, lstm_1_f_b_ih, lstm_1_f_b_hh,
           lstm_1_b_w_ih, lstm_1_b_w_hh, lstm_1_b_b_ih, lstm_1_b_b_hh,
           lin1_w, lin1_b, lin2_w, lin2_b, lin3_w, lin3_b):
    seq = SEQ
    x = pos.reshape(NB, seq, 3).astype(jnp.float32)

    h0 = _layer_scan(x, lstm_0_f_w_ih, lstm_0_f_w_hh, lstm_0_f_b_ih,
                     lstm_0_f_b_hh, lstm_0_b_w_ih, lstm_0_b_w_hh,
                     lstm_0_b_b_ih, lstm_0_b_b_hh)
    h01 = _unscan(h0)
    h1 = _layer_scan(h01, lstm_1_f_w_ih, lstm_1_f_w_hh, lstm_1_f_b_ih,
                     lstm_1_f_b_hh, lstm_1_b_w_ih, lstm_1_b_w_hh,
                     lstm_1_b_b_ih, lstm_1_b_b_hh)
    xlin = _unscan(h1).reshape(NB, 2 * seq)

    kt = 4
    tk = 2 * seq // kt
    y = pl.pallas_call(
        functools.partial(_mlp_kernel, kt=kt),
        out_shape=jax.ShapeDtypeStruct((NB, 1), jnp.float32),
        grid=(kt,),
        in_specs=[
            pl.BlockSpec((NB, tk), lambda k: (0, k)),
            pl.BlockSpec((tk, 2048), lambda k: (k, 0)),
            pl.BlockSpec((1, 2048), lambda k: (0, 0)),
            pl.BlockSpec((2048, 512), lambda k: (0, 0)),
            pl.BlockSpec((1, 512), lambda k: (0, 0)),
            pl.BlockSpec((512, 1), lambda k: (0, 0)),
            pl.BlockSpec((1, 1), lambda k: (0, 0)),
        ],
        out_specs=pl.BlockSpec((NB, 1), lambda k: (0, 0)),
        scratch_shapes=[pltpu.VMEM((NB, 2048), jnp.float32)],
        compiler_params=pltpu.CompilerParams(
            dimension_semantics=("arbitrary",),
            vmem_limit_bytes=50 * 1024 * 1024),
    )(xlin, lin1_w, lin1_b.reshape(1, -1), lin2_w,
      lin2_b.reshape(1, -1), lin3_w, lin3_b.reshape(1, -1))
    return y
